# baseline (speedup 1.0000x reference)
import functools
import math

import jax
import jax.numpy as jnp
from jax import lax
from jax.experimental import pallas as pl
from jax.experimental.pallas import tpu as pltpu
from jax.experimental.pallas import tpu_sc as plsc

B_TOK = 4096
T_SEQ = 200
D_EMBED = 64
VOCAB = 1000
SCALE = math.sqrt(float(D_EMBED))

NUM_CORES = 2
NUM_SUBCORES = 16
NUM_WORKERS = NUM_CORES * NUM_SUBCORES

LANES = 16
BPW = B_TOK // NUM_WORKERS
CI_GROUPS = BPW // LANES
SUB = D_EMBED // 8
KGRP = B_TOK // 128


def _make_lookup():
    mesh = plsc.VectorSubcoreMesh(
        core_axis_name="c", subcore_axis_name="s",
        num_cores=NUM_CORES, num_subcores=NUM_SUBCORES,
    )

    @functools.partial(
        pl.kernel,
        out_type=jax.ShapeDtypeStruct((T_SEQ, SUB, KGRP, 8, 128), jnp.float32),
        mesh=mesh,
        scratch_types=[
            pltpu.VMEM((65024,), jnp.float32),
            pltpu.VMEM((T_SEQ, BPW), jnp.int32),
            pltpu.VMEM((2, SUB, 1, 8, 128), jnp.float32),
            pltpu.SemaphoreType.DMA((2,)),
        ],
        compiler_params=pltpu.CompilerParams(
            use_tc_tiling_on_sc=False, needs_layout_passes=False),
    )
    def lookup_kernel(xt_hbm, tbl_hbm, out_hbm, tbl_v, idx_v, qbuf, sem):
        wid = lax.axis_index("s") * NUM_CORES + lax.axis_index("c")
        b0 = wid * BPW

        pltpu.sync_copy(tbl_hbm, tbl_v.at[pl.ds(0, VOCAB * (D_EMBED + 1))])
        pltpu.sync_copy(xt_hbm.at[:, pl.ds(b0, BPW)], idx_v)

        def scale_body(i, carry):
            off = i * (8 * LANES)
            for u in range(8):
                sl = pl.ds(off + u * LANES, LANES)
                tbl_v[sl] = tbl_v[sl] * SCALE
            return carry

        lax.fori_loop(0, 65024 // (8 * LANES), scale_body, 0)

        tbl_flat = tbl_v

        def compute_t(t, buf):
            for ci in range(CI_GROUPS):
                idxvec = idx_v[t, pl.ds(ci * LANES, LANES)]
                base = idxvec * (D_EMBED + 1)

                @plsc.parallel_loop(0, D_EMBED, unroll=16)
                def d_body(d):
                    val = plsc.load_gather(tbl_flat, [base + d])
                    qbuf[buf, d >> 3, 0, d & 7, pl.ds(ci * LANES, LANES)] = val

        def store_copy(t, buf):
            return pltpu.make_async_copy(
                qbuf.at[buf], out_hbm.at[t, :, pl.ds(wid, 1)], sem.at[buf])

        def pair_body(g, carry):
            for buf in range(2):
                t = g * 2 + buf

                @pl.when(g > 0)
                def _():
                    store_copy(t - 2, buf).wait()

                compute_t(t, buf)
                store_copy(t, buf).start()
            return carry

        lax.fori_loop(0, T_SEQ // 2, pair_body, 0)

        for buf in range(2):
            store_copy(T_SEQ - 2 + buf, buf).wait()

    return lookup_kernel


_lookup = _make_lookup()


@jax.jit
def kernel(x, table):
    xt = jnp.transpose(x)
    tbl_pad = jnp.pad(table, ((0, 0), (0, 1))).reshape(-1)
    q = _lookup(xt, tbl_pad)
    return q.transpose((2, 4, 0, 1, 3)).reshape(B_TOK, T_SEQ, D_EMBED)


# speedup vs baseline: 1.0160x; 1.0160x over previous
import functools
import math

import jax
import jax.numpy as jnp
from jax import lax
from jax.experimental import pallas as pl
from jax.experimental.pallas import tpu as pltpu
from jax.experimental.pallas import tpu_sc as plsc

B_TOK = 4096
T_SEQ = 200
D_EMBED = 64
VOCAB = 1000
SCALE = math.sqrt(float(D_EMBED))

NUM_CORES = 2
NUM_SUBCORES = 16
NUM_WORKERS = NUM_CORES * NUM_SUBCORES

LANES = 16
BPW = B_TOK // NUM_WORKERS
CI_GROUPS = BPW // LANES
SUB = D_EMBED // 8
KGRP = B_TOK // 128


def _make_lookup():
    mesh = plsc.VectorSubcoreMesh(
        core_axis_name="c", subcore_axis_name="s",
        num_cores=NUM_CORES, num_subcores=NUM_SUBCORES,
    )

    @functools.partial(
        pl.kernel,
        out_type=jax.ShapeDtypeStruct((T_SEQ, SUB, KGRP, 8, 128), jnp.float32),
        mesh=mesh,
        scratch_types=[
            pltpu.VMEM((65024,), jnp.float32),
            pltpu.VMEM((T_SEQ, BPW), jnp.int32),
            pltpu.VMEM((2, SUB, 1, 8, 128), jnp.float32),
            pltpu.SemaphoreType.DMA((2,)),
        ],
        compiler_params=pltpu.CompilerParams(
            use_tc_tiling_on_sc=False, needs_layout_passes=False),
    )
    def lookup_kernel(xt_hbm, tbl_hbm, out_hbm, tbl_v, idx_v, qbuf, sem):
        wid = lax.axis_index("s") * NUM_CORES + lax.axis_index("c")
        b0 = wid * BPW

        pltpu.sync_copy(tbl_hbm, tbl_v.at[pl.ds(0, VOCAB * (D_EMBED + 1))])
        pltpu.sync_copy(xt_hbm.at[:, pl.ds(b0, BPW)], idx_v)

        def scale_body(i, carry):
            off = i * (8 * LANES)
            for u in range(8):
                sl = pl.ds(off + u * LANES, LANES)
                tbl_v[sl] = tbl_v[sl] * SCALE
            return carry

        lax.fori_loop(0, 65024 // (8 * LANES), scale_body, 0)

        tbl_flat = tbl_v

        def compute_t(t, buf):
            for ci in range(CI_GROUPS):
                idxvec = idx_v[t, pl.ds(ci * LANES, LANES)]
                base = idxvec * (D_EMBED + 1)

                @plsc.parallel_loop(0, D_EMBED, unroll=8)
                def d_body(d):
                    val = plsc.load_gather(tbl_flat, [base + d])
                    qbuf[buf, d >> 3, 0, d & 7, pl.ds(ci * LANES, LANES)] = val

        def store_copy(t, buf):
            return pltpu.make_async_copy(
                qbuf.at[buf], out_hbm.at[t, :, pl.ds(wid, 1)], sem.at[buf])

        def pair_body(g, carry):
            for buf in range(2):
                t = g * 2 + buf

                @pl.when(g > 0)
                def _():
                    store_copy(t - 2, buf).wait()

                compute_t(t, buf)
                store_copy(t, buf).start()
            return carry

        lax.fori_loop(0, T_SEQ // 2, pair_body, 0)

        for buf in range(2):
            store_copy(T_SEQ - 2 + buf, buf).wait()

    return lookup_kernel


_lookup = _make_lookup()


@jax.jit
def kernel(x, table):
    xt = jnp.transpose(x)
    tbl_pad = jnp.pad(table, ((0, 0), (0, 1))).reshape(-1)
    q = _lookup(xt, tbl_pad)
    return q.transpose((2, 4, 0, 1, 3)).reshape(B_TOK, T_SEQ, D_EMBED)
